# baseline (speedup 1.0000x reference)
import functools

import jax
import jax.numpy as jnp
from jax import lax
from jax.experimental import pallas as pl
from jax.experimental.pallas import tpu as pltpu
from jax.experimental.pallas import tpu_sc as plsc

VOCAB = 100000
EMBED = 64
BATCH = 4096
HIST = 50
NC = 2
NS = 16
NBLK = VOCAB // 128
TAIL = VOCAB - NBLK * 128
TAIL_TILE = NBLK % NS


def _make_kernel():
    mesh = plsc.VectorSubcoreMesh(core_axis_name="c", subcore_axis_name="s")

    @functools.partial(
        pl.kernel,
        mesh=mesh,
        out_type=jax.ShapeDtypeStruct((HIST, EMBED, BATCH), jnp.float32),
        scratch_types=[
            pltpu.HBM((NC, VOCAB, 128), jnp.float32),
            pltpu.VMEM((EMBED, 128), jnp.float32),
            pltpu.VMEM((128, 128), jnp.float32),
            pltpu.VMEM((EMBED, TAIL), jnp.float32),
            pltpu.VMEM((TAIL, 128), jnp.float32),
            pltpu.VMEM((HIST, 128), jnp.int32),
            pltpu.VMEM((128, 128), jnp.float32),
            pltpu.VMEM((EMBED, 128), jnp.float32),
            pltpu.SemaphoreType.DMA,
        ],
        compiler_params=pltpu.CompilerParams(needs_layout_passes=False),
    )
    def fused(table_t, idx_t, out, tscr, blk, tblk, blkt, tblkt, idxv, rows,
              obuf, gsem):
        cid = lax.axis_index("c")
        sid = lax.axis_index("s")
        wid = cid * NS + sid

        iotas = [jax.lax.iota(jnp.int32, 16) + 16 * q for q in range(8)]

        nblk_mine = (NBLK - sid + NS - 1) // NS

        def p1_body(k, carry):
            j = sid + k * NS
            pltpu.sync_copy(table_t.at[:, pl.ds(j * 128, 128)], blk)

            @plsc.parallel_loop(0, 128 * (EMBED // 16), step=1, unroll=16)
            def tr_body(i):
                r = i // (EMBED // 16)
                q16 = (i % (EMBED // 16)) * 16
                cols = jnp.full((16,), 0, jnp.int32) + r
                qi = jax.lax.iota(jnp.int32, 16) + q16
                tblk[r, pl.ds(q16, 16)] = plsc.load_gather(blk, [qi, cols])
            pltpu.sync_copy(tblk, tscr.at[cid, pl.ds(j * 128, 128), :])
            return carry

        lax.fori_loop(0, nblk_mine, p1_body, 0)

        @pl.when(sid == TAIL_TILE)
        def _():
            pltpu.sync_copy(table_t.at[:, pl.ds(NBLK * 128, TAIL)], blkt)

            @plsc.parallel_loop(0, TAIL, step=1, unroll=8)
            def trt_body(r):
                for q in range(EMBED // 16):
                    cols = jnp.full((16,), 0, jnp.int32) + r
                    tblkt[r, pl.ds(16 * q, 16)] = plsc.load_gather(
                        blkt, [iotas[q], cols])
            pltpu.sync_copy(tblkt, tscr.at[cid, pl.ds(NBLK * 128, TAIL), :])

        plsc.subcore_barrier()

        b0 = wid * 128
        pltpu.sync_copy(idx_t.at[:, pl.ds(b0, 128)], idxv)

        def h_body(h, carry):
            pltpu.async_copy(tscr.at[cid].at[idxv.at[h]], rows, gsem).wait()

            @plsc.parallel_loop(0, EMBED * (128 // 16), step=1, unroll=16)
            def e_body(i):
                e = i // (128 // 16)
                q16 = (i % (128 // 16)) * 16
                cols = jnp.full((16,), 0, jnp.int32) + e
                qi = jax.lax.iota(jnp.int32, 16) + q16
                obuf[e, pl.ds(q16, 16)] = plsc.load_gather(rows, [qi, cols])
            pltpu.sync_copy(obuf, out.at[h, :, pl.ds(b0, 128)])
            return carry

        lax.fori_loop(0, HIST, h_body, 0)

    return fused


_fused = _make_kernel()


def kernel(indices, table):
    out_phys = _fused(table.T, indices.T)
    return out_phys.transpose(2, 0, 1)


# speedup vs baseline: 2.5885x; 2.5885x over previous
import functools

import jax
import jax.numpy as jnp
from jax import lax
from jax.experimental import pallas as pl
from jax.experimental.pallas import tpu as pltpu
from jax.experimental.pallas import tpu_sc as plsc

VOCAB = 100000
EMBED = 64
BATCH = 4096
HIST = 50
B = BATCH * HIST
NC = 2
NS = 16
NW = NC * NS
B_PER_W = B // NW
CHUNK = 128
NCHUNK = B_PER_W // CHUNK
NBUF = 10
DEPTH = 6


def _make_gather():
    mesh = plsc.VectorSubcoreMesh(core_axis_name="c", subcore_axis_name="s")

    @functools.partial(
        pl.kernel,
        mesh=mesh,
        out_type=jax.ShapeDtypeStruct((B, EMBED), jnp.float32),
        scratch_types=[
            pltpu.VMEM((B_PER_W,), jnp.int32),
            pltpu.VMEM((NBUF, CHUNK, EMBED), jnp.float32),
        ] + [pltpu.SemaphoreType.DMA] * (2 * NBUF),
        compiler_params=pltpu.CompilerParams(use_tc_tiling_on_sc=False),
    )
    def gather_kernel(idx_hbm, table_hbm, out_hbm, idx_v, rows_v, *sems):
        gsems = sems[:NBUF]
        ssems = sems[NBUF:]
        wid = lax.axis_index("s") * NC + lax.axis_index("c")
        base = wid * B_PER_W
        pltpu.sync_copy(idx_hbm.at[pl.ds(base, B_PER_W)], idx_v)

        def g_copy(c, b):
            return pltpu.make_async_copy(
                table_hbm.at[idx_v.at[pl.ds(c * CHUNK, CHUNK)]],
                rows_v.at[b], gsems[b])

        def s_copy(c, b):
            return pltpu.make_async_copy(
                rows_v.at[b],
                out_hbm.at[pl.ds(base + c * CHUNK, CHUNK)], ssems[b])

        for c in range(DEPTH):
            g_copy(c, c % NBUF).start()

        def body(o, carry):
            c0 = o * NBUF
            for j in range(NBUF):
                c = c0 + j
                g_copy(c, j).wait()
                s_copy(c, j).start()
                cn = c + DEPTH
                b2 = (j + DEPTH) % NBUF

                @pl.when(cn < NCHUNK)
                def _():
                    @pl.when(cn >= NBUF)
                    def _():
                        s_copy(cn - NBUF, b2).wait()

                    g_copy(cn, b2).start()
            return carry

        lax.fori_loop(0, NCHUNK // NBUF, body, 0)

        for j in range(NBUF):
            s_copy(NCHUNK - NBUF + j, j).wait()

    return gather_kernel


_gather = _make_gather()


def kernel(indices, table):
    idx_flat = indices.reshape(B).astype(jnp.int32)
    out = _gather(idx_flat, table)
    return out.reshape(BATCH, HIST, EMBED)
